# Initial kernel scaffold; baseline (speedup 1.0000x reference)
import jax
import jax.numpy as jnp
from jax.experimental import pallas as pl


def kernel(x, edge_index, W1, b1, W2, b2):
    raise NotImplementedError("write your pallas kernel here")


# speedup vs baseline: 11.7303x; 11.7303x over previous
import functools

import jax
import jax.numpy as jnp
from jax import lax
from jax.experimental import pallas as pl
from jax.experimental.pallas import tpu as pltpu
from jax.experimental.pallas import tpu_sc as plsc

N = 10000
D = 128
E = 320000
NC = 2
NS = 16
NW = NC * NS
CHUNK = 128
CPT = -(-E // (NW * CHUNK))
EPT = CPT * CHUNK
EP = EPT * NW
ROWS = 10240
RPT = ROWS // NS
BR = 1024
T_INV = 5.0


def _deg_body(dst_hbm, out_hbm, dst_v, hist_v):
    c = lax.axis_index("c")
    s = lax.axis_index("s")
    wid = c * NS + s

    def zero(i, _):
        hist_v[pl.ds(i * 16, 16)] = jnp.zeros((16,), jnp.float32)
        return 0

    lax.fori_loop(0, ROWS // 16, zero, 0)
    pltpu.sync_copy(dst_hbm.at[wid], dst_v)
    ones = jnp.ones((16,), jnp.float32)

    def chunk(j, _):
        for k in range(CHUNK // 16):
            idx = dst_v[j, pl.ds(k * 16, 16)]
            plsc.addupdate_scatter(hist_v, [idx], ones)
        return 0

    lax.fori_loop(0, CPT, chunk, 0)
    pltpu.sync_copy(hist_v, out_hbm.at[wid])


def _scatter_body(g_hbm, src_hbm, dst_hbm, out_hbm, src_v, dst_v, buf, acc_sh,
                  sem):
    c = lax.axis_index("c")
    s = lax.axis_index("s")
    wid = c * NS + s

    def zero(j, _):
        for k in range(D // 16):
            buf[j, pl.ds(k * 16, 16)] = jnp.zeros((16,), jnp.float32)
        return 0

    lax.fori_loop(0, CHUNK, zero, 0)
    for r in range(RPT // CHUNK):
        pltpu.sync_copy(buf, acc_sh.at[pl.ds(s * RPT + r * CHUNK, CHUNK)])
    pltpu.sync_copy(src_hbm.at[wid], src_v)
    pltpu.sync_copy(dst_hbm.at[wid], dst_v)
    plsc.subcore_barrier()

    def chunk(j, _):
        pltpu.async_copy(g_hbm.at[src_v.at[j]], buf, sem).wait()
        pltpu.sync_copy(buf, acc_sh.at[dst_v.at[j]], add=True)
        return 0

    lax.fori_loop(0, CPT, chunk, 0)
    plsc.subcore_barrier()
    pltpu.sync_copy(acc_sh.at[pl.ds(s * RPT, RPT)],
                    out_hbm.at[c, pl.ds(s * RPT, RPT)])


_SC_MESH = plsc.VectorSubcoreMesh(core_axis_name="c", subcore_axis_name="s")

_deg_kernel = functools.partial(
    pl.kernel,
    out_type=jax.ShapeDtypeStruct((NW, ROWS), jnp.float32),
    mesh=_SC_MESH,
    scratch_types=[
        pltpu.VMEM((CPT, CHUNK), jnp.int32),
        pltpu.VMEM((ROWS,), jnp.float32),
    ],
    compiler_params=pltpu.CompilerParams(needs_layout_passes=False),
)(_deg_body)

_scatter_kernel = functools.partial(
    pl.kernel,
    out_type=jax.ShapeDtypeStruct((NC, ROWS, D), jnp.float32),
    mesh=_SC_MESH,
    scratch_types=[
        pltpu.VMEM((CPT, CHUNK), jnp.int32),
        pltpu.VMEM((CPT, CHUNK), jnp.int32),
        pltpu.VMEM((CHUNK, D), jnp.float32),
        pltpu.VMEM_SHARED((ROWS, D), jnp.float32),
        pltpu.SemaphoreType.DMA,
    ],
)(_scatter_body)


def _tc1_body(dall_ref, x_ref, w1_ref, g_ref, dis_ref):
    deg = jnp.sum(dall_ref[...], axis=0) + 1.0
    dis = lax.rsqrt(deg)
    h = jnp.dot(x_ref[...], w1_ref[...], preferred_element_type=jnp.float32)
    g_ref[...] = dis * h
    dis_ref[...] = dis


def _tc2_body(p_ref, g_ref, dis_ref, b1_ref, w2_ref, g2_ref):
    dis = dis_ref[...]
    agg = dis * (jnp.sum(p_ref[...], axis=0) + g_ref[...])
    h1 = jnp.maximum(agg + b1_ref[...], 0.0)
    g2_ref[...] = dis * jnp.dot(h1, w2_ref[...],
                                preferred_element_type=jnp.float32)


def _tc3_body(q_ref, g2_ref, dis_ref, b2_ref, out_ref):
    o = dis_ref[...] * (jnp.sum(q_ref[...], axis=0) + g2_ref[...]) + b2_ref[...]
    z = o * T_INV
    m = jnp.max(z, axis=1, keepdims=True)
    lse = jnp.log(jnp.sum(jnp.exp(z - m), axis=1, keepdims=True)) + m
    out_ref[...] = z - lse


_GRID = ROWS // BR


def _row_block(i):
    return (i, 0)


_tc1 = pl.pallas_call(
    _tc1_body,
    grid=(_GRID,),
    in_specs=[
        pl.BlockSpec((NW, BR, 1), lambda i: (0, i, 0)),
        pl.BlockSpec((BR, D), _row_block),
        pl.BlockSpec((D, D), lambda i: (0, 0)),
    ],
    out_specs=[
        pl.BlockSpec((BR, D), _row_block),
        pl.BlockSpec((BR, 1), _row_block),
    ],
    out_shape=[
        jax.ShapeDtypeStruct((ROWS, D), jnp.float32),
        jax.ShapeDtypeStruct((ROWS, 1), jnp.float32),
    ],
)

_tc2 = pl.pallas_call(
    _tc2_body,
    grid=(_GRID,),
    in_specs=[
        pl.BlockSpec((NC, BR, D), lambda i: (0, i, 0)),
        pl.BlockSpec((BR, D), _row_block),
        pl.BlockSpec((BR, 1), _row_block),
        pl.BlockSpec((1, D), lambda i: (0, 0)),
        pl.BlockSpec((D, D), lambda i: (0, 0)),
    ],
    out_specs=pl.BlockSpec((BR, D), _row_block),
    out_shape=jax.ShapeDtypeStruct((ROWS, D), jnp.float32),
)

_tc3 = pl.pallas_call(
    _tc3_body,
    grid=(_GRID,),
    in_specs=[
        pl.BlockSpec((NC, BR, D), lambda i: (0, i, 0)),
        pl.BlockSpec((BR, D), _row_block),
        pl.BlockSpec((BR, 1), _row_block),
        pl.BlockSpec((1, D), lambda i: (0, 0)),
    ],
    out_specs=pl.BlockSpec((BR, D), _row_block),
    out_shape=jax.ShapeDtypeStruct((ROWS, D), jnp.float32),
)


def kernel(x, edge_index, W1, b1, W2, b2):
    ei = edge_index.astype(jnp.int32)
    pad = EP - E
    src = jnp.concatenate([ei[0], jnp.zeros((pad,), jnp.int32)])
    dst = jnp.concatenate([ei[1], jnp.full((pad,), N, jnp.int32)])
    src_r = src.reshape(NW, CPT, CHUNK)
    dst_r = dst.reshape(NW, CPT, CHUNK)
    xp = jnp.pad(x, ((0, ROWS - N), (0, 0)))

    deg_parts = _deg_kernel(dst_r)
    dall = deg_parts[:, :, None]
    g, dis = _tc1(dall, xp, W1)
    p = _scatter_kernel(g, src_r, dst_r)
    g2 = _tc2(p, g, dis, b1.reshape(1, D), W2)
    q = _scatter_kernel(g2, src_r, dst_r)
    out = _tc3(q, g2, dis, b2.reshape(1, D))
    return out[:N]


# speedup vs baseline: 13.9701x; 1.1909x over previous
import functools

import jax
import jax.numpy as jnp
from jax import lax
from jax.experimental import pallas as pl
from jax.experimental.pallas import tpu as pltpu
from jax.experimental.pallas import tpu_sc as plsc

N = 10000
D = 128
E = 320000
NC = 2
NS = 16
NW = NC * NS
CHUNK = 96
CPT = -(-E // (NW * CHUNK))
EPT = CPT * CHUNK
EP = EPT * NW
ROWS = 10240
RPT = ROWS // NS
BR = 1024
T_INV = 5.0


def _deg_body(dst_hbm, out_hbm, dst_v, hist_v):
    c = lax.axis_index("c")
    s = lax.axis_index("s")
    wid = c * NS + s

    def zero(i, _):
        hist_v[pl.ds(i * 16, 16)] = jnp.zeros((16,), jnp.float32)
        return 0

    lax.fori_loop(0, ROWS // 16, zero, 0)
    pltpu.sync_copy(dst_hbm.at[wid], dst_v)
    ones = jnp.ones((16,), jnp.float32)

    def chunk(j, _):
        for k in range(CHUNK // 16):
            idx = dst_v[j, pl.ds(k * 16, 16)]
            plsc.addupdate_scatter(hist_v, [idx], ones)
        return 0

    lax.fori_loop(0, CPT, chunk, 0)
    pltpu.sync_copy(hist_v, out_hbm.at[wid])


def _scatter_body(g_hbm, src_hbm, dst_hbm, out_hbm, src_v, dst_v, buf,
                  acc_sh, gsem):
    c = lax.axis_index("c")
    s = lax.axis_index("s")
    wid = c * NS + s
    half0 = buf.at[pl.ds(0, CHUNK)]

    def zero(j, _):
        for k in range(D // 16):
            buf[j, pl.ds(k * 16, 16)] = jnp.zeros((16,), jnp.float32)
        return 0

    lax.fori_loop(0, 128, zero, 0)
    for r in range(RPT // 128):
        pltpu.sync_copy(buf.at[pl.ds(0, 128)],
                        acc_sh.at[pl.ds(s * RPT + r * 128, 128)])
    pltpu.sync_copy(src_hbm.at[wid], src_v)
    pltpu.sync_copy(dst_hbm.at[wid], dst_v)
    plsc.subcore_barrier()

    pltpu.sync_copy(g_hbm.at[src_v.at[pl.ds(0, CHUNK)]], half0)

    def step(j, _):
        nxt = jnp.minimum(j + 1, CPT - 1)
        off_n = ((j + 1) % 2) * CHUNK
        off = (j % 2) * CHUNK
        cp = pltpu.async_copy(g_hbm.at[src_v.at[pl.ds(nxt * CHUNK, CHUNK)]],
                              buf.at[pl.ds(off_n, CHUNK)], gsem)
        pltpu.sync_copy(buf.at[pl.ds(off, CHUNK)], acc_sh.at[dst_v.at[j]],
                        add=True)
        cp.wait()
        return 0

    lax.fori_loop(0, CPT, step, 0)
    plsc.subcore_barrier()
    pltpu.sync_copy(acc_sh.at[pl.ds(s * RPT, RPT)],
                    out_hbm.at[c, pl.ds(s * RPT, RPT)])


_SC_MESH = plsc.VectorSubcoreMesh(core_axis_name="c", subcore_axis_name="s")

_deg_kernel = functools.partial(
    pl.kernel,
    out_type=jax.ShapeDtypeStruct((NW, ROWS), jnp.float32),
    mesh=_SC_MESH,
    scratch_types=[
        pltpu.VMEM((CPT, CHUNK), jnp.int32),
        pltpu.VMEM((ROWS,), jnp.float32),
    ],
    compiler_params=pltpu.CompilerParams(needs_layout_passes=False),
)(_deg_body)

_scatter_kernel = functools.partial(
    pl.kernel,
    out_type=jax.ShapeDtypeStruct((NC, ROWS, D), jnp.float32),
    mesh=_SC_MESH,
    scratch_types=[
        pltpu.VMEM((EPT,), jnp.int32),
        pltpu.VMEM((CPT, CHUNK), jnp.int32),
        pltpu.VMEM((2 * CHUNK, D), jnp.float32),
        pltpu.VMEM_SHARED((ROWS, D), jnp.float32),
        pltpu.SemaphoreType.DMA,
    ],
)(_scatter_body)


def _tc1_body(dall_ref, x_ref, w1_ref, g_ref, dis_ref):
    deg = jnp.sum(dall_ref[...], axis=0) + 1.0
    dis = lax.rsqrt(deg)
    h = jnp.dot(x_ref[...], w1_ref[...], preferred_element_type=jnp.float32)
    g_ref[...] = dis * h
    dis_ref[...] = dis


def _tc2_body(p_ref, g_ref, dis_ref, b1_ref, w2_ref, g2_ref):
    dis = dis_ref[...]
    agg = dis * (jnp.sum(p_ref[...], axis=0) + g_ref[...])
    h1 = jnp.maximum(agg + b1_ref[...], 0.0)
    g2_ref[...] = dis * jnp.dot(h1, w2_ref[...],
                                preferred_element_type=jnp.float32)


def _tc3_body(q_ref, g2_ref, dis_ref, b2_ref, out_ref):
    o = dis_ref[...] * (jnp.sum(q_ref[...], axis=0) + g2_ref[...]) + b2_ref[...]
    z = o * T_INV
    m = jnp.max(z, axis=1, keepdims=True)
    lse = jnp.log(jnp.sum(jnp.exp(z - m), axis=1, keepdims=True)) + m
    out_ref[...] = z - lse


_GRID = ROWS // BR


def _row_block(i):
    return (i, 0)


_tc1 = pl.pallas_call(
    _tc1_body,
    grid=(_GRID,),
    in_specs=[
        pl.BlockSpec((NW, BR, 1), lambda i: (0, i, 0)),
        pl.BlockSpec((BR, D), _row_block),
        pl.BlockSpec((D, D), lambda i: (0, 0)),
    ],
    out_specs=[
        pl.BlockSpec((BR, D), _row_block),
        pl.BlockSpec((BR, 1), _row_block),
    ],
    out_shape=[
        jax.ShapeDtypeStruct((ROWS, D), jnp.float32),
        jax.ShapeDtypeStruct((ROWS, 1), jnp.float32),
    ],
)

_tc2 = pl.pallas_call(
    _tc2_body,
    grid=(_GRID,),
    in_specs=[
        pl.BlockSpec((NC, BR, D), lambda i: (0, i, 0)),
        pl.BlockSpec((BR, D), _row_block),
        pl.BlockSpec((BR, 1), _row_block),
        pl.BlockSpec((1, D), lambda i: (0, 0)),
        pl.BlockSpec((D, D), lambda i: (0, 0)),
    ],
    out_specs=pl.BlockSpec((BR, D), _row_block),
    out_shape=jax.ShapeDtypeStruct((ROWS, D), jnp.float32),
)

_tc3 = pl.pallas_call(
    _tc3_body,
    grid=(_GRID,),
    in_specs=[
        pl.BlockSpec((NC, BR, D), lambda i: (0, i, 0)),
        pl.BlockSpec((BR, D), _row_block),
        pl.BlockSpec((BR, 1), _row_block),
        pl.BlockSpec((1, D), lambda i: (0, 0)),
    ],
    out_specs=pl.BlockSpec((BR, D), _row_block),
    out_shape=jax.ShapeDtypeStruct((ROWS, D), jnp.float32),
)


def kernel(x, edge_index, W1, b1, W2, b2):
    ei = edge_index.astype(jnp.int32)
    pad = EP - E
    src = jnp.concatenate([ei[0], jnp.zeros((pad,), jnp.int32)])
    dst = jnp.concatenate([ei[1], jnp.full((pad,), N, jnp.int32)])
    src_r = src.reshape(NW, EPT)
    dst_r = dst.reshape(NW, CPT, CHUNK)
    xp = jnp.pad(x, ((0, ROWS - N), (0, 0)))

    deg_parts = _deg_kernel(dst_r)
    dall = deg_parts[:, :, None]
    g, dis = _tc1(dall, xp, W1)
    p = _scatter_kernel(g, src_r, dst_r)
    g2 = _tc2(p, g, dis, b1.reshape(1, D), W2)
    q = _scatter_kernel(g2, src_r, dst_r)
    out = _tc3(q, g2, dis, b2.reshape(1, D))
    return out[:N]
